# baseline (speedup 1.0000x reference)
import numpy as np

import jax
import jax.numpy as jnp
from jax import lax
from jax.experimental import pallas as pl
from jax.experimental.pallas import tpu as pltpu
from jax.experimental.pallas import tpu_sc as plsc

NC = 2
NS = 16
L = 16
NW = NC * NS

ROWS = 4096
COLS = 8192
RPW = ROWS // NW
RG = 4
NG = RPW // RG
U = 2
CHUNKS = COLS // L

BIG = np.int32(COLS)


def _argmin_body(x_hbm, out_hbm, buf_a, buf_b, res, sem_a, sem_b):
    c = lax.axis_index("c")
    s = lax.axis_index("s")
    wid = s * NC + c
    base_row = wid * RPW

    lane = lax.iota(jnp.int32, L)

    def start(g, buf, sem):
        src = x_hbm.at[pl.ds((base_row + g * RG) * COLS, RG * COLS)]
        return pltpu.async_copy(src, buf, sem)

    def process_block(buf, acc, row0):
        vmin0 = [jnp.full((L,), jnp.inf, jnp.float32) for _ in range(RG)]
        vidx0 = [jnp.zeros((L,), jnp.int32) for _ in range(RG)]
        idx0 = [lane + k * L for k in range(U)]

        def step(i, carry):
            vmins, vidxs, idxs = carry
            col0 = i * (U * L)
            vmins = list(vmins)
            vidxs = list(vidxs)
            for k in range(U):
                for r in range(RG):
                    v = buf[pl.ds(r * COLS + col0 + k * L, L)]
                    m = v < vmins[r]
                    vmins[r] = jnp.where(m, v, vmins[r])
                    vidxs[r] = jnp.where(m, idxs[k], vidxs[r])
            idxs = tuple(ix + U * L for ix in idxs)
            return tuple(vmins), tuple(vidxs), idxs

        vmins, vidxs, _ = lax.fori_loop(
            0, CHUNKS // U, step, (tuple(vmin0), tuple(vidx0), tuple(idx0))
        )

        dnums = lax.GatherDimensionNumbers(
            offset_dims=(), collapsed_slice_dims=(0,), start_index_map=(0,)
        )

        def shuffle(v, perm):
            return lax.gather(
                v, perm[:, None], dnums, slice_sizes=(1,),
                mode=lax.GatherScatterMode.PROMISE_IN_BOUNDS,
            )

        perms = [lane ^ (1 << d) for d in range(4)]
        for r in range(RG):
            vm = vmins[r]
            for p in perms:
                vm = jnp.minimum(vm, shuffle(vm, p))
            cand = jnp.where(vmins[r] == vm, vidxs[r], jnp.full((L,), BIG))
            for p in perms:
                cand = jnp.minimum(cand, shuffle(cand, p))
            slot = (row0 + r) % L
            acc = jnp.where(lane == slot, cand, acc)
        return acc

    acc = jnp.zeros((L,), jnp.int32)
    cp = start(0, buf_a, sem_a)
    for g in range(NG):
        buf = buf_a if g % 2 == 0 else buf_b
        if g + 1 < NG:
            ncp = start(g + 1, buf_b if g % 2 == 0 else buf_a,
                        sem_b if g % 2 == 0 else sem_a)
        cp.wait()
        row0 = g * RG
        acc = process_block(buf, acc, row0)
        if (row0 + RG) % L == 0:
            res[pl.ds(row0 + RG - L, L)] = acc
            acc = jnp.zeros((L,), jnp.int32)
        if g + 1 < NG:
            cp = ncp

    pltpu.sync_copy(res, out_hbm.at[pl.ds(base_row, RPW)])


@jax.jit
def _argmin_sc(x_flat):
    call = pl.kernel(
        _argmin_body,
        out_type=jax.ShapeDtypeStruct((ROWS,), jnp.int32),
        mesh=plsc.VectorSubcoreMesh(core_axis_name="c", subcore_axis_name="s"),
        scratch_types=[
            pltpu.VMEM((RG * COLS,), jnp.float32),
            pltpu.VMEM((RG * COLS,), jnp.float32),
            pltpu.VMEM((RPW,), jnp.int32),
            pltpu.SemaphoreType.DMA,
            pltpu.SemaphoreType.DMA,
        ],
    )
    return call(x_flat)


def kernel(x):
    n, h, k = x.shape
    out = _argmin_sc(x.reshape(n * h * k))
    return out.reshape(n, h)


# speedup vs baseline: 2.4754x; 2.4754x over previous
import numpy as np

import jax
import jax.numpy as jnp
from jax import lax
from jax.experimental import pallas as pl
from jax.experimental.pallas import tpu as pltpu
from jax.experimental.pallas import tpu_sc as plsc

NC = 2
NS = 16
L = 16
NW = NC * NS

N0 = 128
N1 = 32
COLS = 8192
SR = 8
HALF = COLS // 2
TILES = HALF // 128
KPT = 128 // L

N_TC = 80
N_SC = N0 - N_TC
SC_ROWS = N_SC * N1
SPW = SC_ROWS // (SR * NW)
RPW = SPW * SR

BIG = np.int32(COLS)


def _sc_body(x_hbm, out_hbm, buf_a, buf_b, res, sem_a, sem_b):
    c = lax.axis_index("c")
    s = lax.axis_index("s")
    wid = s * NC + c

    lane = lax.iota(jnp.int32, L)

    def src(strip, half):
        n = N_TC + strip // (N1 // SR)
        h0 = (strip % (N1 // SR)) * SR
        return x_hbm.at[n, pl.ds(h0, SR), pl.ds(half * HALF, HALF)]

    RG = 4

    def process_half(buf, col_base, vmins, vidxs):
        vmins, vidxs = list(vmins), list(vidxs)
        for r0 in range(0, SR, RG):
            def step(t, carry, r0=r0):
                vm, vi = list(carry[0]), list(carry[1])
                for k in range(KPT):
                    col = t * 128 + k * L
                    idx = col_base + col + lane
                    for j in range(RG):
                        v = buf[r0 + j, pl.ds(col, L)]
                        m = v < vm[j]
                        vm[j] = jnp.where(m, v, vm[j])
                        vi[j] = jnp.where(m, idx, vi[j])
                return tuple(vm), tuple(vi)

            vm, vi = lax.fori_loop(
                0, TILES, step,
                (tuple(vmins[r0:r0 + RG]), tuple(vidxs[r0:r0 + RG])),
            )
            vmins[r0:r0 + RG] = list(vm)
            vidxs[r0:r0 + RG] = list(vi)
        return tuple(vmins), tuple(vidxs)

    dnums = lax.GatherDimensionNumbers(
        offset_dims=(), collapsed_slice_dims=(0,), start_index_map=(0,)
    )

    def shuffle(v, perm):
        return lax.gather(
            v, perm[:, None], dnums, slice_sizes=(1,),
            mode=lax.GatherScatterMode.PROMISE_IN_BOUNDS,
        )

    perms = [lane ^ (1 << d) for d in range(4)]

    def epilogue(vmins, vidxs, acc, slot0):
        for r in range(SR):
            vm = vmins[r]
            for p in perms:
                vm = jnp.minimum(vm, shuffle(vm, p))
            cand = jnp.where(vmins[r] == vm, vidxs[r], jnp.full((L,), BIG))
            for p in perms:
                cand = jnp.minimum(cand, shuffle(cand, p))
            acc = jnp.where(lane == (slot0 + r), cand, acc)
        return acc

    strip0 = wid * SPW
    pltpu.async_copy(src(strip0, 0), buf_a, sem_a)

    def strip_body(i, acc):
        strip = strip0 + i
        cp_b = pltpu.async_copy(src(strip, 1), buf_b, sem_b)
        pltpu.make_async_copy(src(strip, 0), buf_a, sem_a).wait()

        vmins = tuple(jnp.full((L,), jnp.inf, jnp.float32) for _ in range(SR))
        vidxs = tuple(jnp.zeros((L,), jnp.int32) for _ in range(SR))
        vmins, vidxs = process_half(buf_a, 0, vmins, vidxs)

        @pl.when(i + 1 < SPW)
        def _():
            pltpu.async_copy(src(strip + 1, 0), buf_a, sem_a)

        cp_b.wait()
        vmins, vidxs = process_half(buf_b, HALF, vmins, vidxs)

        acc = epilogue(vmins, vidxs, acc, (i % 2) * SR)

        @pl.when(i % 2 == 1)
        def _():
            res[pl.ds((i - 1) * SR, L)] = acc

        return jnp.where(i % 2 == 1, jnp.zeros((L,), jnp.int32), acc)

    lax.fori_loop(0, SPW, strip_body, jnp.zeros((L,), jnp.int32))

    pltpu.sync_copy(res, out_hbm.at[pl.ds(wid * RPW, RPW)])


def _sc_call(x):
    call = pl.kernel(
        _sc_body,
        out_type=jax.ShapeDtypeStruct((SC_ROWS,), jnp.int32),
        mesh=plsc.VectorSubcoreMesh(core_axis_name="c", subcore_axis_name="s"),
        scratch_types=[
            pltpu.VMEM((SR, HALF), jnp.float32),
            pltpu.VMEM((SR, HALF), jnp.float32),
            pltpu.VMEM((RPW,), jnp.int32),
            pltpu.SemaphoreType.DMA,
            pltpu.SemaphoreType.DMA,
        ],
        compiler_params=pltpu.CompilerParams(use_tc_tiling_on_sc=True),
    )
    return call(x)


BN = 8


CW = 512
NCH = COLS // CW


def _tc_block(x_ref, o_ref):
    def step(c, carry):
        vmin, vchunk = carry
        v = x_ref[:, :, pl.ds(c * CW, CW)]
        m = v < vmin
        vmin = jnp.where(m, v, vmin)
        vchunk = jnp.where(m, c, vchunk)
        return vmin, vchunk

    vmin0 = jnp.full((BN, N1, CW), jnp.inf, jnp.float32)
    vchunk0 = jnp.zeros((BN, N1, CW), jnp.int32)
    vmin, vchunk = lax.fori_loop(0, NCH, step, (vmin0, vchunk0))

    mrow = jnp.min(vmin, axis=2, keepdims=True)
    iota = lax.broadcasted_iota(jnp.int32, (BN, N1, CW), 2)
    idx = vchunk * CW + iota
    cand = jnp.where(vmin == mrow, idx, COLS)
    o_ref[...] = jnp.min(cand, axis=2).astype(jnp.int32)


def _tc_call(x):
    return pl.pallas_call(
        _tc_block,
        grid=(N_TC // BN,),
        in_specs=[pl.BlockSpec((BN, N1, COLS), lambda i: (i, 0, 0))],
        out_specs=pl.BlockSpec((BN, N1), lambda i: (i, 0)),
        out_shape=jax.ShapeDtypeStruct((N_TC, N1), jnp.int32),
    )(x)


@jax.jit
def _argmin_hybrid(x):
    out_sc = _sc_call(x)
    out_tc = _tc_call(x)
    return jnp.concatenate([out_tc, out_sc.reshape(N_SC, N1)], axis=0)


def kernel(x):
    return _argmin_hybrid(x)


# speedup vs baseline: 2.8867x; 1.1662x over previous
import numpy as np

import jax
import jax.numpy as jnp
from jax import lax
from jax.experimental import pallas as pl
from jax.experimental.pallas import tpu as pltpu
from jax.experimental.pallas import tpu_sc as plsc

NC = 2
NS = 16
L = 16
NW = NC * NS

N0 = 128
N1 = 32
COLS = 8192
SR = 8
HALF = COLS // 2
TILES = HALF // 128
KPT = 128 // L

N_TC = 80
N_SC = N0 - N_TC
SC_ROWS = N_SC * N1
SPW = SC_ROWS // (SR * NW)
RPW = SPW * SR

BIG = np.int32(COLS)


def _sc_body(x_hbm, out_hbm, buf_a, buf_b, res, sem_a, sem_b):
    c = lax.axis_index("c")
    s = lax.axis_index("s")
    wid = s * NC + c

    lane = lax.iota(jnp.int32, L)

    def src(strip, half):
        n = N_TC + strip // (N1 // SR)
        h0 = (strip % (N1 // SR)) * SR
        return x_hbm.at[n, pl.ds(h0, SR), pl.ds(half * HALF, HALF)]

    RG = 4

    def process_half(buf, col_base, vmins, vidxs):
        vmins, vidxs = list(vmins), list(vidxs)
        for r0 in range(0, SR, RG):
            def step(t, carry, r0=r0):
                vm, vi = list(carry[0]), list(carry[1])
                for k in range(KPT):
                    col = t * 128 + k * L
                    idx = col_base + col + lane
                    for j in range(RG):
                        v = buf[r0 + j, pl.ds(col, L)]
                        m = v < vm[j]
                        vm[j] = jnp.where(m, v, vm[j])
                        vi[j] = jnp.where(m, idx, vi[j])
                return tuple(vm), tuple(vi)

            vm, vi = lax.fori_loop(
                0, TILES, step,
                (tuple(vmins[r0:r0 + RG]), tuple(vidxs[r0:r0 + RG])),
            )
            vmins[r0:r0 + RG] = list(vm)
            vidxs[r0:r0 + RG] = list(vi)
        return tuple(vmins), tuple(vidxs)

    dnums = lax.GatherDimensionNumbers(
        offset_dims=(), collapsed_slice_dims=(0,), start_index_map=(0,)
    )

    def shuffle(v, perm):
        return lax.gather(
            v, perm[:, None], dnums, slice_sizes=(1,),
            mode=lax.GatherScatterMode.PROMISE_IN_BOUNDS,
        )

    perms = [lane ^ (1 << d) for d in range(4)]

    def epilogue(vmins, vidxs, acc, slot0):
        for r in range(SR):
            vm = vmins[r]
            for p in perms:
                vm = jnp.minimum(vm, shuffle(vm, p))
            cand = jnp.where(vmins[r] == vm, vidxs[r], jnp.full((L,), BIG))
            for p in perms:
                cand = jnp.minimum(cand, shuffle(cand, p))
            acc = jnp.where(lane == (slot0 + r), cand, acc)
        return acc

    strip0 = wid * SPW
    pltpu.async_copy(src(strip0, 0), buf_a, sem_a)

    def strip_body(i, acc):
        strip = strip0 + i
        cp_b = pltpu.async_copy(src(strip, 1), buf_b, sem_b)
        pltpu.make_async_copy(src(strip, 0), buf_a, sem_a).wait()

        vmins = tuple(jnp.full((L,), jnp.inf, jnp.float32) for _ in range(SR))
        vidxs = tuple(jnp.zeros((L,), jnp.int32) for _ in range(SR))
        vmins, vidxs = process_half(buf_a, 0, vmins, vidxs)

        @pl.when(i + 1 < SPW)
        def _():
            pltpu.async_copy(src(strip + 1, 0), buf_a, sem_a)

        cp_b.wait()
        vmins, vidxs = process_half(buf_b, HALF, vmins, vidxs)

        acc = epilogue(vmins, vidxs, acc, (i % 2) * SR)

        @pl.when(i % 2 == 1)
        def _():
            res[pl.ds((i - 1) * SR, L)] = acc

        return jnp.where(i % 2 == 1, jnp.zeros((L,), jnp.int32), acc)

    lax.fori_loop(0, SPW, strip_body, jnp.zeros((L,), jnp.int32))

    pltpu.sync_copy(res, out_hbm.at[pl.ds(wid * RPW, RPW)])


def _sc_call(x):
    call = pl.kernel(
        _sc_body,
        out_type=jax.ShapeDtypeStruct((SC_ROWS,), jnp.int32),
        mesh=plsc.VectorSubcoreMesh(core_axis_name="c", subcore_axis_name="s"),
        scratch_types=[
            pltpu.VMEM((SR, HALF), jnp.float32),
            pltpu.VMEM((SR, HALF), jnp.float32),
            pltpu.VMEM((RPW,), jnp.int32),
            pltpu.SemaphoreType.DMA,
            pltpu.SemaphoreType.DMA,
        ],
        compiler_params=pltpu.CompilerParams(use_tc_tiling_on_sc=True),
    )
    return call(x)


BN = 8


def _tc_block(x_ref, o_ref):
    xb = x_ref[...]
    m = jnp.min(xb, axis=2, keepdims=True)
    iota = lax.broadcasted_iota(jnp.int32, xb.shape, 2)
    cand = jnp.where(xb == m, iota, COLS)
    o_ref[...] = jnp.min(cand, axis=2).astype(jnp.int32)


def _tc_call(x):
    return pl.pallas_call(
        _tc_block,
        grid=(N_TC // BN,),
        in_specs=[pl.BlockSpec((BN, N1, COLS), lambda i: (i, 0, 0))],
        out_specs=pl.BlockSpec((BN, N1), lambda i: (i, 0)),
        out_shape=jax.ShapeDtypeStruct((N_TC, N1), jnp.int32),
    )(x)


@jax.jit
def _argmin_hybrid(x):
    out_sc = _sc_call(x)
    out_tc = _tc_call(x)
    return jnp.concatenate([out_tc, out_sc.reshape(N_SC, N1)], axis=0)


def kernel(x):
    return _argmin_hybrid(x)
